# baseline (speedup 1.0000x reference)
import functools

import jax
import jax.numpy as jnp
from jax import lax
from jax.experimental import pallas as pl
from jax.experimental.pallas import tpu as pltpu
from jax.experimental.pallas import tpu_sc as plsc

N = 65536
K = 256
M = 64
L = 16
NC = 2
NS = 16
NW = NC * NS
RPW = N // NW
RB = 128
NBLK = RPW // RB
CPR = K // L


def _sc_body(x_hbm, wt_hbm, b_hbm, o_hbm, xbuf, obuf, wtbuf, bbuf):
    wid = lax.axis_index("s") * NC + lax.axis_index("c")
    pltpu.sync_copy(wt_hbm, wtbuf)
    pltpu.sync_copy(b_hbm, bbuf)
    lanes = lax.iota(jnp.int32, L)
    bvecs = [bbuf[pl.ds(j * L, L)] for j in range(M // L)]

    def do_block(b, _):
        rowbase = wid * RPW + b * RB
        pltpu.sync_copy(x_hbm.at[pl.ds(rowbase, RB), :], xbuf)

        def initrow(r, _):
            for j in range(M // L):
                obuf[r, pl.ds(j * L, L)] = bvecs[j]
            return 0

        lax.fori_loop(0, RB, initrow, 0, unroll=4)

        def do_chunk(i, _):
            r = i // CPR
            cbase = (i % CPR) * L
            v = xbuf[r, pl.ds(cbase, L)]
            mi = jnp.where(v != 0.0, 1, 0)
            t = mi | mi[(lanes + 1) & (L - 1)]
            t = t | t[(lanes + 2) & (L - 1)]
            t = t | t[(lanes + 4) & (L - 1)]
            t = t | t[(lanes + 8) & (L - 1)]
            flag = t[0]

            @pl.when(flag != 0)
            def _():
                for l in range(L):
                    val = v[l]

                    @pl.when(val != 0.0)
                    def _():
                        c = cbase + l
                        for j in range(M // L):
                            sl = pl.ds(j * L, L)
                            obuf[r, sl] = obuf[r, sl] + val * wtbuf[c, sl]

            return 0

        lax.fori_loop(0, RB * CPR, do_chunk, 0)
        pltpu.sync_copy(obuf, o_hbm.at[pl.ds(rowbase, RB), :])
        return 0

    lax.fori_loop(0, NBLK, do_block, 0)


@jax.jit
def _sc_linear(x, wt, bias):
    mesh = plsc.VectorSubcoreMesh(core_axis_name="c", subcore_axis_name="s")
    f = functools.partial(
        pl.kernel,
        mesh=mesh,
        out_type=jax.ShapeDtypeStruct((N, M), jnp.float32),
        scratch_types=[
            pltpu.VMEM((RB, K), jnp.float32),
            pltpu.VMEM((RB, M), jnp.float32),
            pltpu.VMEM((K, M), jnp.float32),
            pltpu.VMEM((M,), jnp.float32),
        ],
    )(_sc_body)
    return f(x, wt, bias)


def kernel(input, weight, bias):
    return _sc_linear(input, weight.T, bias)


# speedup vs baseline: 98.5224x; 98.5224x over previous
import jax
import jax.numpy as jnp
from jax import lax
from jax.experimental import pallas as pl
from jax.experimental.pallas import tpu as pltpu

N = 65536
K = 256
M = 64
BLOCK_N = 8192
NBUF = 4
NSTEPS = N // BLOCK_N


def _mm_body(x_hbm, w_ref, b_ref, o_hbm, *rest):
    xbufs = rest[:NBUF]
    obufs = rest[NBUF : 2 * NBUF]
    insems, outsems = rest[2 * NBUF], rest[2 * NBUF + 1]

    def in_copy(i, s):
        return pltpu.make_async_copy(
            x_hbm.at[pl.ds(i * BLOCK_N, BLOCK_N), :], xbufs[s], insems.at[s]
        )

    def out_copy(i, s):
        return pltpu.make_async_copy(
            obufs[s], o_hbm.at[:, pl.ds(i * BLOCK_N, BLOCK_N)], outsems.at[s]
        )

    for i in range(NBUF):
        in_copy(i, i).start()
    for i in range(NSTEPS):
        s = i % NBUF
        in_copy(i, s).wait()
        if i >= NBUF:
            out_copy(i - NBUF, s).wait()
        obufs[s][...] = (
            lax.dot_general(
                w_ref[...],
                xbufs[s][...],
                (((1,), (1,)), ((), ())),
                preferred_element_type=jnp.float32,
            )
            + b_ref[...]
        )
        out_copy(i, s).start()
        if i + NBUF < NSTEPS:
            in_copy(i + NBUF, s).start()
    for i in range(NSTEPS - NBUF, NSTEPS):
        out_copy(i, i % NBUF).wait()


@jax.jit
def _matmul_t(x, w, bias_col):
    return pl.pallas_call(
        _mm_body,
        in_specs=[
            pl.BlockSpec(memory_space=pl.ANY),
            pl.BlockSpec(memory_space=pltpu.VMEM),
            pl.BlockSpec(memory_space=pltpu.VMEM),
        ],
        out_specs=pl.BlockSpec(memory_space=pl.ANY),
        out_shape=jax.ShapeDtypeStruct((M, N), jnp.float32),
        scratch_shapes=(
            [pltpu.VMEM((BLOCK_N, K), jnp.float32) for _ in range(NBUF)]
            + [pltpu.VMEM((M, BLOCK_N), jnp.float32) for _ in range(NBUF)]
            + [
                pltpu.SemaphoreType.DMA((NBUF,)),
                pltpu.SemaphoreType.DMA((NBUF,)),
            ]
        ),
    )(x, w, bias_col)


def kernel(input, weight, bias):
    out_t = _matmul_t(input, weight, bias.reshape(M, 1))
    return out_t.T


# speedup vs baseline: 100.7067x; 1.0222x over previous
import jax
import jax.numpy as jnp
from jax import lax
from jax.experimental import pallas as pl
from jax.experimental.pallas import tpu as pltpu

N = 65536
K = 256
M = 64
BLOCK_N = 4096
NBUF = 8
NSTEPS = N // BLOCK_N


def _mm_body(x_hbm, w_ref, b_ref, o_hbm, *rest):
    xbufs = rest[:NBUF]
    obufs = rest[NBUF : 2 * NBUF]
    insems, outsems = rest[2 * NBUF], rest[2 * NBUF + 1]

    def in_copy(i, s):
        return pltpu.make_async_copy(
            x_hbm.at[pl.ds(i * BLOCK_N, BLOCK_N), :], xbufs[s], insems.at[s]
        )

    def out_copy(i, s):
        return pltpu.make_async_copy(
            obufs[s], o_hbm.at[:, pl.ds(i * BLOCK_N, BLOCK_N)], outsems.at[s]
        )

    for i in range(NBUF):
        in_copy(i, i).start()
    for i in range(NSTEPS):
        s = i % NBUF
        in_copy(i, s).wait()
        if i >= NBUF:
            out_copy(i - NBUF, s).wait()
        obufs[s][...] = (
            lax.dot_general(
                w_ref[...],
                xbufs[s][...],
                (((1,), (1,)), ((), ())),
                preferred_element_type=jnp.float32,
            )
            + b_ref[...]
        )
        out_copy(i, s).start()
        if i + NBUF < NSTEPS:
            in_copy(i + NBUF, s).start()
    for i in range(NSTEPS - NBUF, NSTEPS):
        out_copy(i, i % NBUF).wait()


@jax.jit
def _matmul_t(x, w, bias_col):
    return pl.pallas_call(
        _mm_body,
        in_specs=[
            pl.BlockSpec(memory_space=pl.ANY),
            pl.BlockSpec(memory_space=pltpu.VMEM),
            pl.BlockSpec(memory_space=pltpu.VMEM),
        ],
        out_specs=pl.BlockSpec(memory_space=pl.ANY),
        out_shape=jax.ShapeDtypeStruct((M, N), jnp.float32),
        scratch_shapes=(
            [pltpu.VMEM((BLOCK_N, K), jnp.float32) for _ in range(NBUF)]
            + [pltpu.VMEM((M, BLOCK_N), jnp.float32) for _ in range(NBUF)]
            + [
                pltpu.SemaphoreType.DMA((NBUF,)),
                pltpu.SemaphoreType.DMA((NBUF,)),
            ]
        ),
    )(x, w, bias_col)


def kernel(input, weight, bias):
    out_t = _matmul_t(input, weight, bias.reshape(M, 1))
    return out_t.T


# speedup vs baseline: 101.3941x; 1.0068x over previous
import jax
import jax.numpy as jnp
from jax import lax
from jax.experimental import pallas as pl
from jax.experimental.pallas import tpu as pltpu

N = 65536
K = 256
M = 64
BLOCK_N = 2048
NBUF = 16
NSTEPS = N // BLOCK_N


def _mm_body(x_hbm, w_ref, b_ref, o_hbm, *rest):
    xbufs = rest[:NBUF]
    obufs = rest[NBUF : 2 * NBUF]
    insems, outsems = rest[2 * NBUF], rest[2 * NBUF + 1]

    def in_copy(i, s):
        return pltpu.make_async_copy(
            x_hbm.at[pl.ds(i * BLOCK_N, BLOCK_N), :], xbufs[s], insems.at[s]
        )

    def out_copy(i, s):
        return pltpu.make_async_copy(
            obufs[s], o_hbm.at[:, pl.ds(i * BLOCK_N, BLOCK_N)], outsems.at[s]
        )

    for i in range(NBUF):
        in_copy(i, i).start()
    for i in range(NSTEPS):
        s = i % NBUF
        in_copy(i, s).wait()
        if i >= NBUF:
            out_copy(i - NBUF, s).wait()
        obufs[s][...] = (
            lax.dot_general(
                w_ref[...],
                xbufs[s][...],
                (((1,), (1,)), ((), ())),
                preferred_element_type=jnp.float32,
            )
            + b_ref[...]
        )
        out_copy(i, s).start()
        if i + NBUF < NSTEPS:
            in_copy(i + NBUF, s).start()
    for i in range(NSTEPS - NBUF, NSTEPS):
        out_copy(i, i % NBUF).wait()


@jax.jit
def _matmul_t(x, w, bias_col):
    return pl.pallas_call(
        _mm_body,
        in_specs=[
            pl.BlockSpec(memory_space=pl.ANY),
            pl.BlockSpec(memory_space=pltpu.VMEM),
            pl.BlockSpec(memory_space=pltpu.VMEM),
        ],
        out_specs=pl.BlockSpec(memory_space=pl.ANY),
        out_shape=jax.ShapeDtypeStruct((M, N), jnp.float32),
        scratch_shapes=(
            [pltpu.VMEM((BLOCK_N, K), jnp.float32) for _ in range(NBUF)]
            + [pltpu.VMEM((M, BLOCK_N), jnp.float32) for _ in range(NBUF)]
            + [
                pltpu.SemaphoreType.DMA((NBUF,)),
                pltpu.SemaphoreType.DMA((NBUF,)),
            ]
        ),
    )(x, w, bias_col)


def kernel(input, weight, bias):
    out_t = _matmul_t(input, weight, bias.reshape(M, 1))
    return out_t.T
